# baseline (device time: 30807 ns/iter reference)
import functools

import jax
import jax.numpy as jnp
from jax import lax
from jax.experimental import pallas as pl
from jax.experimental.pallas import tpu as pltpu

N_DEV = 16
N_ROUNDS = 4
N_EXPERTS = 32
E_PER_DEV = 2
CAPACITY = 6


def kernel(x, router_W, route_idx, expert_W):
    del router_W
    n, d = x.shape
    h = expert_W.shape[-1]

    def body(x_ref, idx_ref, w_ref, out_ref, comm_ref, send_sems, recv_sems):
        my_i = lax.axis_index("i")
        partners = [my_i ^ (1 << r) for r in range(N_ROUNDS)]

        barrier_sem = pltpu.get_barrier_semaphore()
        for p in partners:
            pl.semaphore_signal(
                barrier_sem, inc=1,
                device_id=(p,), device_id_type=pl.DeviceIdType.MESH,
            )
        pl.semaphore_wait(barrier_sem, N_ROUNDS)

        idx = idx_ref[:, :]
        e_iota = lax.broadcasted_iota(jnp.int32, (n, N_EXPERTS), 1)
        onehot = (idx == e_iota).astype(jnp.float32)
        row = lax.broadcasted_iota(jnp.int32, (n, n), 0)
        col = lax.broadcasted_iota(jnp.int32, (n, n), 1)
        lower_tri = (col <= row).astype(jnp.bfloat16)
        cum = jnp.dot(
            lower_tri, onehot.astype(jnp.bfloat16),
            preferred_element_type=jnp.float32,
        )
        rank_incl = jnp.sum(cum * onehot, axis=1, keepdims=True)
        keep = rank_incl <= CAPACITY

        xb = x_ref[:, :]
        e0 = my_i * E_PER_DEV
        acc = jnp.zeros((n, h), jnp.float32)
        for k in range(E_PER_DEV):
            m = ((idx == e0 + k) & keep).astype(jnp.float32)
            xm = (xb * m).astype(jnp.bfloat16)
            wk = w_ref[k].astype(jnp.bfloat16)
            acc = acc + jnp.dot(xm, wk, preferred_element_type=jnp.float32)
        out_ref[:, :] = acc

        for r in range(N_ROUNDS):
            rdma = pltpu.make_async_remote_copy(
                src_ref=out_ref,
                dst_ref=comm_ref.at[r],
                send_sem=send_sems.at[r],
                recv_sem=recv_sems.at[r],
                device_id=(partners[r],),
                device_id_type=pl.DeviceIdType.MESH,
            )
            rdma.start()
            rdma.wait()
            out_ref[:, :] = out_ref[:, :] + comm_ref[r]

        @functools.partial(
            pl.run_scoped, exit_sem=pltpu.SemaphoreType.REGULAR
        )
        def _(exit_sem):
            for p in partners:
                pl.semaphore_signal(
                    exit_sem, inc=1,
                    device_id=(p,), device_id_type=pl.DeviceIdType.MESH,
                )
            pl.semaphore_wait(exit_sem, N_ROUNDS)

    return pl.pallas_call(
        body,
        out_shape=jax.ShapeDtypeStruct((n, h), jnp.float32),
        in_specs=[
            pl.BlockSpec(memory_space=pltpu.VMEM),
            pl.BlockSpec(memory_space=pltpu.VMEM),
            pl.BlockSpec(memory_space=pltpu.VMEM),
        ],
        out_specs=pl.BlockSpec(memory_space=pltpu.VMEM),
        scratch_shapes=[
            pltpu.VMEM((N_ROUNDS, n, h), jnp.float32),
            pltpu.SemaphoreType.DMA((N_ROUNDS,)),
            pltpu.SemaphoreType.DMA((N_ROUNDS,)),
        ],
        compiler_params=pltpu.CompilerParams(collective_id=0),
    )(x, route_idx, expert_W)


# device time: 23699 ns/iter; 1.2999x vs baseline; 1.2999x over previous
import functools

import jax
import jax.numpy as jnp
from jax import lax
from jax.experimental import pallas as pl
from jax.experimental.pallas import tpu as pltpu

N_DEV = 16
N_ROUNDS = 4
N_EXPERTS = 32
E_PER_DEV = 2
CAPACITY = 6


def kernel(x, router_W, route_idx, expert_W):
    del router_W
    n, d = x.shape
    h = expert_W.shape[-1]

    def body(x_ref, idx_ref, w_ref, out_ref, acc_ref, comm_ref,
             send_sems, recv_sems):
        my_i = lax.axis_index("i")
        partners = [my_i ^ (1 << r) for r in range(N_ROUNDS)]

        barrier_sem = pltpu.get_barrier_semaphore()
        for p in partners:
            pl.semaphore_signal(
                barrier_sem, inc=1,
                device_id=(p,), device_id_type=pl.DeviceIdType.MESH,
            )
        pl.semaphore_wait(barrier_sem, N_ROUNDS)

        idx = idx_ref[:, :]
        e_iota = lax.broadcasted_iota(jnp.int32, (n, N_EXPERTS), 1)
        onehot = (idx == e_iota).astype(jnp.float32)
        row = lax.broadcasted_iota(jnp.int32, (n, n), 0)
        col = lax.broadcasted_iota(jnp.int32, (n, n), 1)
        lower_tri = (col <= row).astype(jnp.bfloat16)
        cum = jnp.dot(
            lower_tri, onehot.astype(jnp.bfloat16),
            preferred_element_type=jnp.float32,
        )
        rank_incl = jnp.sum(cum * onehot, axis=1, keepdims=True)
        keep = rank_incl <= CAPACITY

        xb = x_ref[:, :]
        e0 = my_i * E_PER_DEV
        acc = jnp.zeros((n, h), jnp.float32)
        for k in range(E_PER_DEV):
            m = ((idx == e0 + k) & keep).astype(jnp.float32)
            xm = (xb * m).astype(jnp.bfloat16)
            wk = w_ref[k].astype(jnp.bfloat16)
            acc = acc + jnp.dot(xm, wk, preferred_element_type=jnp.float32)
        acc_ref[:, :] = acc.astype(jnp.bfloat16)

        for r in range(N_ROUNDS):
            rdma = pltpu.make_async_remote_copy(
                src_ref=acc_ref,
                dst_ref=comm_ref.at[r],
                send_sem=send_sems.at[r],
                recv_sem=recv_sems.at[r],
                device_id=(partners[r],),
                device_id_type=pl.DeviceIdType.MESH,
            )
            rdma.start()
            rdma.wait()
            acc_ref[:, :] = acc_ref[:, :] + comm_ref[r]

        out_ref[:, :] = acc_ref[:, :].astype(jnp.float32)

        @functools.partial(
            pl.run_scoped, exit_sem=pltpu.SemaphoreType.REGULAR
        )
        def _(exit_sem):
            for p in partners:
                pl.semaphore_signal(
                    exit_sem, inc=1,
                    device_id=(p,), device_id_type=pl.DeviceIdType.MESH,
                )
            pl.semaphore_wait(exit_sem, N_ROUNDS)

    return pl.pallas_call(
        body,
        out_shape=jax.ShapeDtypeStruct((n, h), jnp.float32),
        in_specs=[
            pl.BlockSpec(memory_space=pltpu.VMEM),
            pl.BlockSpec(memory_space=pltpu.VMEM),
            pl.BlockSpec(memory_space=pltpu.VMEM),
        ],
        out_specs=pl.BlockSpec(memory_space=pltpu.VMEM),
        scratch_shapes=[
            pltpu.VMEM((n, h), jnp.bfloat16),
            pltpu.VMEM((N_ROUNDS, n, h), jnp.bfloat16),
            pltpu.SemaphoreType.DMA((N_ROUNDS,)),
            pltpu.SemaphoreType.DMA((N_ROUNDS,)),
        ],
        compiler_params=pltpu.CompilerParams(collective_id=0),
    )(x, route_idx, expert_W)


# device time: 17662 ns/iter; 1.7443x vs baseline; 1.3418x over previous
import functools

import jax
import jax.numpy as jnp
from jax import lax
from jax.experimental import pallas as pl
from jax.experimental.pallas import tpu as pltpu

N_DEV = 16
N_EXPERTS = 32
E_PER_DEV = 2
CAPACITY = 6
SLOTS = 8
BLK = E_PER_DEV * SLOTS


def kernel(x, router_W, route_idx, expert_W):
    del router_W
    n, d = x.shape
    h = expert_W.shape[-1]

    def body(x_ref, idx_ref, w_ref, out_ref, comm_ref, send_sem, recv_sem):
        my_i = lax.axis_index("i")

        barrier_sem = pltpu.get_barrier_semaphore()
        for o in range(1, N_DEV):
            pl.semaphore_signal(
                barrier_sem, inc=1,
                device_id=(my_i ^ o,), device_id_type=pl.DeviceIdType.MESH,
            )
        pl.semaphore_wait(barrier_sem, N_DEV - 1)

        idx = idx_ref[:, :]
        e_iota = lax.broadcasted_iota(jnp.int32, (n, N_EXPERTS), 1)
        onehot = (idx == e_iota).astype(jnp.bfloat16)
        row = lax.broadcasted_iota(jnp.int32, (n, n), 0)
        col = lax.broadcasted_iota(jnp.int32, (n, n), 1)
        lower_tri = (col <= row).astype(jnp.bfloat16)
        cum = jnp.dot(lower_tri, onehot, preferred_element_type=jnp.float32)
        rank_incl = jnp.sum(
            cum * onehot.astype(jnp.float32), axis=1, keepdims=True
        )
        keep = rank_incl <= CAPACITY
        rank0 = rank_incl.astype(jnp.int32) - 1

        mine = (idx >= E_PER_DEV * my_i) & (idx < E_PER_DEV * (my_i + 1))
        lslot = (idx - E_PER_DEV * my_i) * SLOTS + rank0
        j_iota = lax.broadcasted_iota(jnp.int32, (BLK, n), 0)
        gmat = (
            (lslot.reshape(1, n) == j_iota)
            & keep.reshape(1, n) & mine.reshape(1, n)
        ).astype(jnp.bfloat16)
        xc = jnp.dot(
            gmat, x_ref[:, :].astype(jnp.bfloat16),
            preferred_element_type=jnp.float32,
        ).astype(jnp.bfloat16)

        my_rows = my_i * BLK
        for k in range(E_PER_DEV):
            blk = jnp.dot(
                xc[k * SLOTS:(k + 1) * SLOTS, :],
                w_ref[k].astype(jnp.bfloat16),
                preferred_element_type=jnp.float32,
            ).astype(jnp.bfloat16)
            comm_ref[pl.ds(my_rows + k * SLOTS, SLOTS), :] = blk

        rdmas = []
        for o in range(1, N_DEV):
            rdma = pltpu.make_async_remote_copy(
                src_ref=comm_ref.at[pl.ds(my_rows, BLK), :],
                dst_ref=comm_ref.at[pl.ds(my_rows, BLK), :],
                send_sem=send_sem,
                recv_sem=recv_sem,
                device_id=(my_i ^ o,),
                device_id_type=pl.DeviceIdType.MESH,
            )
            rdma.start()
            rdmas.append(rdma)

        for _ in range(N_DEV - 1):
            rdmas[0].wait_recv()

        gslot = idx * SLOTS + rank0
        g_iota = lax.broadcasted_iota(jnp.int32, (n, N_DEV * BLK), 1)
        pmat = ((gslot == g_iota) & keep).astype(jnp.bfloat16)
        out_ref[:, :] = jnp.dot(
            pmat, comm_ref[:, :], preferred_element_type=jnp.float32
        )

        for rdma in rdmas:
            rdma.wait_send()

        @functools.partial(
            pl.run_scoped, exit_sem=pltpu.SemaphoreType.REGULAR
        )
        def _(exit_sem):
            for o in range(1, N_DEV):
                pl.semaphore_signal(
                    exit_sem, inc=1,
                    device_id=(my_i ^ o,), device_id_type=pl.DeviceIdType.MESH,
                )
            pl.semaphore_wait(exit_sem, N_DEV - 1)

    return pl.pallas_call(
        body,
        out_shape=jax.ShapeDtypeStruct((n, h), jnp.float32),
        in_specs=[
            pl.BlockSpec(memory_space=pltpu.VMEM),
            pl.BlockSpec(memory_space=pltpu.VMEM),
            pl.BlockSpec(memory_space=pltpu.VMEM),
        ],
        out_specs=pl.BlockSpec(memory_space=pltpu.VMEM),
        scratch_shapes=[
            pltpu.VMEM((N_DEV * BLK, h), jnp.bfloat16),
            pltpu.SemaphoreType.DMA,
            pltpu.SemaphoreType.DMA,
        ],
        compiler_params=pltpu.CompilerParams(collective_id=0),
    )(x, route_idx, expert_W)


# device time: 12238 ns/iter; 2.5173x vs baseline; 1.4432x over previous
import jax
import jax.numpy as jnp
from jax import lax
from jax.experimental import pallas as pl
from jax.experimental.pallas import tpu as pltpu

N_DEV = 16
N_EXPERTS = 32
E_PER_DEV = 2
CAPACITY = 6
SLOTS = 8
BLK = E_PER_DEV * SLOTS


def kernel(x, router_W, route_idx, expert_W):
    del router_W
    n, d = x.shape
    h = expert_W.shape[-1]

    def body(x_ref, idx_ref, w_ref, out_ref, comm_ref, send_sem, recv_sem):
        my_i = lax.axis_index("i")

        barrier_sem = pltpu.get_barrier_semaphore()
        for o in range(1, N_DEV):
            pl.semaphore_signal(
                barrier_sem, inc=1,
                device_id=(my_i ^ o,), device_id_type=pl.DeviceIdType.MESH,
            )

        idx = idx_ref[:, :]
        row = lax.broadcasted_iota(jnp.int32, (n, n), 0)
        col = lax.broadcasted_iota(jnp.int32, (n, n), 1)
        same = (idx == idx.reshape(1, n)) & (col <= row)
        rank_incl = jnp.sum(
            same.astype(jnp.float32), axis=1, keepdims=True
        )
        keep = rank_incl <= CAPACITY
        rank0 = rank_incl.astype(jnp.int32) - 1

        mine = (idx >= E_PER_DEV * my_i) & (idx < E_PER_DEV * (my_i + 1))
        lslot = (idx - E_PER_DEV * my_i) * SLOTS + rank0
        j_iota = lax.broadcasted_iota(jnp.int32, (BLK, n), 0)
        gmat = (
            (lslot.reshape(1, n) == j_iota)
            & keep.reshape(1, n) & mine.reshape(1, n)
        ).astype(jnp.bfloat16)
        xc = jnp.dot(
            gmat, x_ref[:, :].astype(jnp.bfloat16),
            preferred_element_type=jnp.float32,
        ).astype(jnp.bfloat16)

        my_rows = my_i * BLK
        for k in range(E_PER_DEV):
            blk = jnp.dot(
                xc[k * SLOTS:(k + 1) * SLOTS, :],
                w_ref[k].astype(jnp.bfloat16),
                preferred_element_type=jnp.float32,
            ).astype(jnp.bfloat16)
            comm_ref[pl.ds(my_rows + k * SLOTS, SLOTS), :] = blk

        pl.semaphore_wait(barrier_sem, N_DEV - 1)

        rdmas = []
        for o in range(1, N_DEV):
            rdma = pltpu.make_async_remote_copy(
                src_ref=comm_ref.at[pl.ds(my_rows, BLK), :],
                dst_ref=comm_ref.at[pl.ds(my_rows, BLK), :],
                send_sem=send_sem,
                recv_sem=recv_sem,
                device_id=(my_i ^ o,),
                device_id_type=pl.DeviceIdType.MESH,
            )
            rdma.start()
            rdmas.append(rdma)

        gslot = idx * SLOTS + rank0
        g_iota = lax.broadcasted_iota(jnp.int32, (n, N_DEV * BLK), 1)
        pmat = ((gslot == g_iota) & keep).astype(jnp.bfloat16)

        for _ in range(N_DEV - 1):
            rdmas[0].wait_recv()

        out_ref[:, :] = jnp.dot(
            pmat, comm_ref[:, :], preferred_element_type=jnp.float32
        )

        for rdma in rdmas:
            rdma.wait_send()

    return pl.pallas_call(
        body,
        out_shape=jax.ShapeDtypeStruct((n, h), jnp.float32),
        in_specs=[
            pl.BlockSpec(memory_space=pltpu.VMEM),
            pl.BlockSpec(memory_space=pltpu.VMEM),
            pl.BlockSpec(memory_space=pltpu.VMEM),
        ],
        out_specs=pl.BlockSpec(memory_space=pltpu.VMEM),
        scratch_shapes=[
            pltpu.VMEM((N_DEV * BLK, h), jnp.bfloat16),
            pltpu.SemaphoreType.DMA,
            pltpu.SemaphoreType.DMA,
        ],
        compiler_params=pltpu.CompilerParams(collective_id=0),
    )(x, route_idx, expert_W)
